# Initial kernel scaffold; baseline (speedup 1.0000x reference)
import jax
import jax.numpy as jnp
from jax.experimental import pallas as pl


def kernel(h, P, coeffs):
    raise NotImplementedError("write your pallas kernel here")


# speedup vs baseline: 1.2244x; 1.2244x over previous
import functools

import jax
import jax.numpy as jnp
from jax import lax
from jax.experimental import pallas as pl
from jax.experimental.pallas import tpu as pltpu
from jax.experimental.pallas import tpu_sc as plsc
import numpy as np

NPTS = 1048576
NGRID = 512
NPROPS = 5
HMIN, HMAX = 1.0e5, 4.0e6
PMIN, PMAX = 1.0e4, 1.0e7
LMIN, LMAX = float(np.log(PMIN)), float(np.log(PMAX))
NCELL = (NGRID - 1) * (NGRID - 1)
NROWS = NPROPS * NCELL

NC, NS = 2, 16
NW = NC * NS
PW = NPTS // NW
B = 512
NCH = PW // B
NSUB = B // 128


def _bin_body(h_ref, p_ref, idx_ref, x_ref, y_ref):
    h = h_ref[...]
    ii = (h - HMIN) / (HMAX - HMIN) * (NGRID - 1)
    i = jnp.clip(ii.astype(jnp.int32), 0, NGRID - 2)
    x = ii - i.astype(ii.dtype)
    L = jnp.log(p_ref[...])
    jj = (L - LMIN) / (LMAX - LMIN) * (NGRID - 1)
    j = jnp.clip(jj.astype(jnp.int32), 0, NGRID - 2)
    y = jj - j.astype(jj.dtype)
    idx_ref[...] = i * (NGRID - 1) + j
    x_ref[...] = x
    y_ref[...] = y


def _binning(h2, p2):
    blk = pl.BlockSpec((128, 1024), lambda g: (g, 0))
    return pl.pallas_call(
        _bin_body,
        grid=(8,),
        in_specs=[blk, blk],
        out_specs=[blk, blk, blk],
        out_shape=[
            jax.ShapeDtypeStruct((1024, 1024), jnp.int32),
            jax.ShapeDtypeStruct((1024, 1024), jnp.float32),
            jax.ShapeDtypeStruct((1024, 1024), jnp.float32),
        ],
    )(h2, p2)


def _sc_body(idx_hbm, x_hbm, y_hbm, table_hbm, out_hbm,
             xv, yv, idxs, rows, outs, sem):
    wid = lax.axis_index("s") * NC + lax.axis_index("c")

    def chunk(g, carry):
        base = wid * PW + g * B
        rowb = wid * (PW // 128) + g * NSUB
        pltpu.sync_copy(x_hbm.at[pl.ds(base, B)], xv)
        pltpu.sync_copy(y_hbm.at[pl.ds(base, B)], yv)
        pltpu.sync_copy(idx_hbm.at[pl.ds(rowb, NSUB)], idxs[0])

        for s in range(NSUB):
            def off(l, c, s=s):
                v = idxs[0][s, pl.ds(l * 16, 16)]
                for p in range(1, NPROPS):
                    idxs[p][s, pl.ds(l * 16, 16)] = v + p * NCELL
                return c
            lax.fori_loop(0, 8, off, carry)

        handles = []
        for p in range(NPROPS):
            for s in range(NSUB):
                handles.append(pltpu.async_copy(
                    table_hbm.at[idxs[p].at[s]],
                    rows[p].at[pl.ds(s * 128, 128)],
                    sem))
        for hd in handles:
            hd.wait()

        iota16 = lax.iota(jnp.int32, 16)

        def qgrp(q, c):
            xq = xv[pl.ds(q * 16, 16)]
            yq = yv[pl.ds(q * 16, 16)]
            x2 = xq * xq
            x3 = x2 * xq
            y2 = yq * yq
            y3 = y2 * yq
            xp = [None, xq, x2, x3]
            yp = [None, yq, y2, y3]
            terms = []
            for n in range(4):
                for m in range(4):
                    if xp[m] is None and yp[n] is None:
                        terms.append(None)
                    elif xp[m] is None:
                        terms.append(yp[n])
                    elif yp[n] is None:
                        terms.append(xp[m])
                    else:
                        terms.append(xp[m] * yp[n])
            rowv = iota16 + q * 16
            for p in range(NPROPS):
                acc = None
                for k in range(16):
                    colv = jnp.full((16,), k, jnp.int32)
                    cf = plsc.load_gather(rows[p], [rowv, colv])
                    if terms[k] is None:
                        acc = cf if acc is None else acc + cf
                    else:
                        acc = acc + cf * terms[k]
                outs[p][pl.ds(q * 16, 16)] = acc
            return c
        lax.fori_loop(0, B // 16, qgrp, carry)

        for p in range(NPROPS):
            pltpu.sync_copy(outs[p], out_hbm.at[pl.ds(p * NPTS + base, B)])
        return carry

    lax.fori_loop(0, NCH, chunk, 0)


@functools.partial(jax.jit, static_argnums=())
def _lookup_poly(idx2d, xf, yf, table):
    mesh = plsc.VectorSubcoreMesh(core_axis_name="c", subcore_axis_name="s")

    def body(idx_hbm, x_hbm, y_hbm, table_hbm, out_hbm, *scratch):
        xv, yv = scratch[0], scratch[1]
        idxs = list(scratch[2:2 + NPROPS])
        rows = list(scratch[2 + NPROPS:2 + 2 * NPROPS])
        outs = list(scratch[2 + 2 * NPROPS:2 + 3 * NPROPS])
        sem = scratch[-1]
        _sc_body(idx_hbm, x_hbm, y_hbm, table_hbm, out_hbm,
                 xv, yv, idxs, rows, outs, sem)

    scratch_types = (
        [pltpu.VMEM((B,), jnp.float32), pltpu.VMEM((B,), jnp.float32)]
        + [pltpu.VMEM((NSUB, 128), jnp.int32) for _ in range(NPROPS)]
        + [pltpu.VMEM((B, 16), jnp.float32) for _ in range(NPROPS)]
        + [pltpu.VMEM((B,), jnp.float32) for _ in range(NPROPS)]
        + [pltpu.SemaphoreType.DMA]
    )
    fn = pl.kernel(
        body,
        out_type=jax.ShapeDtypeStruct((NPROPS * NPTS,), jnp.float32),
        mesh=mesh,
        compiler_params=pltpu.CompilerParams(needs_layout_passes=False,
                                             use_tc_tiling_on_sc=False),
        scratch_types=scratch_types,
    )
    return fn(idx2d, xf, yf, table)


def kernel(h, P, coeffs):
    h2 = h.reshape(1024, 1024)
    p2 = P.reshape(1024, 1024)
    idx2, x2, y2 = _binning(h2, p2)
    idx2d = idx2.reshape(NPTS // 128, 128)
    xf = x2.reshape(NPTS)
    yf = y2.reshape(NPTS)
    table = coeffs.reshape(NROWS, 16)
    out = _lookup_poly(idx2d, xf, yf, table)
    return out.reshape(NPROPS, NPTS)


# speedup vs baseline: 2.5467x; 2.0799x over previous
import functools

import jax
import jax.numpy as jnp
from jax import lax
from jax.experimental import pallas as pl
from jax.experimental.pallas import tpu as pltpu
from jax.experimental.pallas import tpu_sc as plsc
import numpy as np

NPTS = 1048576
NGRID = 512
NPROPS = 5
HMIN, HMAX = 1.0e5, 4.0e6
PMIN, PMAX = 1.0e4, 1.0e7
LMIN, LMAX = float(np.log(PMIN)), float(np.log(PMAX))
NI = NGRID - 1
JSTRIDE = 512
TROWS = NI * JSTRIDE

NC, NS = 2, 16
NW = NC * NS
PW = NPTS // NW
B = 512
NCH = PW // B
NSUB = B // 128
PROWS = NPTS // 128


def _repack_body(c_ref, t_ref):
    a = c_ref[...]
    parts = [a[p, 0] for p in range(NPROPS)]
    row = jnp.concatenate(parts, axis=-1)
    row = jnp.concatenate(
        [row, jnp.zeros((NI, 128 - 16 * NPROPS), jnp.float32)], axis=-1)
    row = jnp.concatenate([row, jnp.zeros((1, 128), jnp.float32)], axis=0)
    t_ref[...] = row


def _repack(coeffs):
    return pl.pallas_call(
        _repack_body,
        grid=(NI,),
        in_specs=[pl.BlockSpec((NPROPS, 1, NI, 16), lambda i: (0, i, 0, 0))],
        out_specs=pl.BlockSpec((JSTRIDE, 128), lambda i: (i, 0)),
        out_shape=jax.ShapeDtypeStruct((TROWS, 128), jnp.float32),
    )(coeffs)


def _bin_body(h_ref, p_ref, idx_ref, x_ref, y_ref):
    h = h_ref[...]
    ii = (h - HMIN) / (HMAX - HMIN) * (NGRID - 1)
    i = jnp.clip(ii.astype(jnp.int32), 0, NGRID - 2)
    x = ii - i.astype(ii.dtype)
    L = jnp.log(p_ref[...])
    jj = (L - LMIN) / (LMAX - LMIN) * (NGRID - 1)
    j = jnp.clip(jj.astype(jnp.int32), 0, NGRID - 2)
    y = jj - j.astype(jj.dtype)
    idx_ref[...] = i * JSTRIDE + j
    x_ref[...] = x
    y_ref[...] = y


def _binning(h2, p2):
    blk = pl.BlockSpec((1024, 128), lambda g: (g, 0))
    return pl.pallas_call(
        _bin_body,
        grid=(8,),
        in_specs=[blk, blk],
        out_specs=[blk, blk, blk],
        out_shape=[
            jax.ShapeDtypeStruct((PROWS, 128), jnp.int32),
            jax.ShapeDtypeStruct((PROWS, 128), jnp.float32),
            jax.ShapeDtypeStruct((PROWS, 128), jnp.float32),
        ],
    )(h2, p2)


def _sc_body(idx_hbm, x_hbm, y_hbm, table_hbm, out_hbm,
             xv, yv, idxv, rows, outs, sem):
    wid = lax.axis_index("s") * NC + lax.axis_index("c")
    iota16 = lax.iota(jnp.int32, 16)

    def chunk(g, carry):
        rowb = wid * (PW // 128) + g * NSUB
        pltpu.sync_copy(x_hbm.at[pl.ds(rowb, NSUB)], xv)
        pltpu.sync_copy(y_hbm.at[pl.ds(rowb, NSUB)], yv)
        pltpu.sync_copy(idx_hbm.at[pl.ds(rowb, NSUB)], idxv)

        handles = []
        for s in range(NSUB):
            handles.append(pltpu.async_copy(
                table_hbm.at[idxv.at[s]],
                rows.at[pl.ds(s * 128, 128)],
                sem))
        for hd in handles:
            hd.wait()

        for s in range(NSUB):
            def qgrp(l, c, s=s):
                xq = xv[s, pl.ds(l * 16, 16)]
                yq = yv[s, pl.ds(l * 16, 16)]
                x2 = xq * xq
                x3 = x2 * xq
                y2 = yq * yq
                y3 = y2 * yq
                xp = [None, xq, x2, x3]
                yp = [None, yq, y2, y3]
                terms = []
                for n in range(4):
                    for m in range(4):
                        if xp[m] is None and yp[n] is None:
                            terms.append(None)
                        elif xp[m] is None:
                            terms.append(yp[n])
                        elif yp[n] is None:
                            terms.append(xp[m])
                        else:
                            terms.append(xp[m] * yp[n])
                rowv = iota16 + (s * 128 + l * 16)
                for p in range(NPROPS):
                    acc = None
                    for k in range(16):
                        colv = jnp.full((16,), p * 16 + k, jnp.int32)
                        cf = plsc.load_gather(rows, [rowv, colv])
                        if terms[k] is None:
                            acc = cf if acc is None else acc + cf
                        else:
                            acc = acc + cf * terms[k]
                    outs[p][s, pl.ds(l * 16, 16)] = acc
                return c
            lax.fori_loop(0, 8, qgrp, carry)

        for p in range(NPROPS):
            pltpu.sync_copy(outs[p],
                            out_hbm.at[pl.ds(p * PROWS + rowb, NSUB)])
        return carry

    lax.fori_loop(0, NCH, chunk, 0)


def _lookup_poly(idx2d, xf, yf, table):
    mesh = plsc.VectorSubcoreMesh(core_axis_name="c", subcore_axis_name="s")

    def body(idx_hbm, x_hbm, y_hbm, table_hbm, out_hbm, *scratch):
        xv, yv, idxv, rows = scratch[0], scratch[1], scratch[2], scratch[3]
        outs = list(scratch[4:4 + NPROPS])
        sem = scratch[-1]
        _sc_body(idx_hbm, x_hbm, y_hbm, table_hbm, out_hbm,
                 xv, yv, idxv, rows, outs, sem)

    scratch_types = (
        [pltpu.VMEM((NSUB, 128), jnp.float32),
         pltpu.VMEM((NSUB, 128), jnp.float32),
         pltpu.VMEM((NSUB, 128), jnp.int32),
         pltpu.VMEM((B, 128), jnp.float32)]
        + [pltpu.VMEM((NSUB, 128), jnp.float32) for _ in range(NPROPS)]
        + [pltpu.SemaphoreType.DMA]
    )
    fn = pl.kernel(
        body,
        out_type=jax.ShapeDtypeStruct((NPROPS * PROWS, 128), jnp.float32),
        mesh=mesh,
        compiler_params=pltpu.CompilerParams(needs_layout_passes=False,
                                             use_tc_tiling_on_sc=False),
        scratch_types=scratch_types,
    )
    return fn(idx2d, xf, yf, table)


def kernel(h, P, coeffs):
    h2 = h.reshape(PROWS, 128)
    p2 = P.reshape(PROWS, 128)
    table = _repack(coeffs)
    idx2, x2, y2 = _binning(h2, p2)
    out2d = _lookup_poly(idx2, x2, y2, table)
    return out2d.reshape(NPROPS, PROWS, 128).reshape(NPROPS, NPTS)


# speedup vs baseline: 2.5538x; 1.0028x over previous
import functools

import jax
import jax.numpy as jnp
from jax import lax
from jax.experimental import pallas as pl
from jax.experimental.pallas import tpu as pltpu
from jax.experimental.pallas import tpu_sc as plsc
import numpy as np

NPTS = 1048576
NGRID = 512
NPROPS = 5
HMIN, HMAX = 1.0e5, 4.0e6
PMIN, PMAX = 1.0e4, 1.0e7
LMIN, LMAX = float(np.log(PMIN)), float(np.log(PMAX))
NI = NGRID - 1
JSTRIDE = 512
TROWS = NI * JSTRIDE

NC, NS = 2, 16
NW = NC * NS
PW = NPTS // NW
B = 512
NCH = PW // B
NSUB = B // 128
PROWS = NPTS // 128


def _repack_body(c_ref, t_ref):
    a = c_ref[...]
    parts = [a[p, 0] for p in range(NPROPS)]
    row = jnp.concatenate(parts, axis=-1)
    row = jnp.concatenate(
        [row, jnp.zeros((NI, 128 - 16 * NPROPS), jnp.float32)], axis=-1)
    row = jnp.concatenate([row, jnp.zeros((1, 128), jnp.float32)], axis=0)
    t_ref[...] = row


def _repack(coeffs):
    return pl.pallas_call(
        _repack_body,
        grid=(NI,),
        in_specs=[pl.BlockSpec((NPROPS, 1, NI, 16), lambda i: (0, i, 0, 0))],
        out_specs=pl.BlockSpec((JSTRIDE, 128), lambda i: (i, 0)),
        out_shape=jax.ShapeDtypeStruct((TROWS, 128), jnp.float32),
    )(coeffs)


_CH = float(np.float32(np.float32(1.0 / (HMAX - HMIN)) * np.float32(NGRID - 1)))
_CL = float(np.float32(np.float32(1.0 / (LMAX - LMIN)) * np.float32(NGRID - 1)))


def _bin_body(h_ref, p_ref, idx_ref, x_ref, y_ref):
    h = h_ref[...]
    ii = (h - HMIN) * _CH
    i = jnp.clip(ii.astype(jnp.int32), 0, NGRID - 2)
    x = ii - i.astype(ii.dtype)
    L = jnp.log(p_ref[...])
    jj = (L - LMIN) * _CL
    j = jnp.clip(jj.astype(jnp.int32), 0, NGRID - 2)
    y = jj - j.astype(jj.dtype)
    idx_ref[...] = i * JSTRIDE + j
    x_ref[...] = x
    y_ref[...] = y


def _binning(h2, p2):
    blk = pl.BlockSpec((1024, 128), lambda g: (g, 0))
    return pl.pallas_call(
        _bin_body,
        grid=(8,),
        in_specs=[blk, blk],
        out_specs=[blk, blk, blk],
        out_shape=[
            jax.ShapeDtypeStruct((PROWS, 128), jnp.int32),
            jax.ShapeDtypeStruct((PROWS, 128), jnp.float32),
            jax.ShapeDtypeStruct((PROWS, 128), jnp.float32),
        ],
    )(h2, p2)


def _sc_body(idx_hbm, x_hbm, y_hbm, table_hbm, out_hbm,
             xv, yv, idxv, rows, outs, sem):
    wid = lax.axis_index("s") * NC + lax.axis_index("c")
    iota16 = lax.iota(jnp.int32, 16)

    def chunk(g, carry):
        rowb = wid * (PW // 128) + g * NSUB
        pltpu.sync_copy(x_hbm.at[pl.ds(rowb, NSUB)], xv)
        pltpu.sync_copy(y_hbm.at[pl.ds(rowb, NSUB)], yv)
        pltpu.sync_copy(idx_hbm.at[pl.ds(rowb, NSUB)], idxv)

        handles = []
        for s in range(NSUB):
            handles.append(pltpu.async_copy(
                table_hbm.at[idxv.at[s]],
                rows.at[pl.ds(s * 128, 128)],
                sem))
        for hd in handles:
            hd.wait()

        for s in range(NSUB):
            def qgrp(l, c, s=s):
                xq = xv[s, pl.ds(l * 16, 16)]
                yq = yv[s, pl.ds(l * 16, 16)]
                x2 = xq * xq
                x3 = x2 * xq
                y2 = yq * yq
                y3 = y2 * yq
                xp = [None, xq, x2, x3]
                yp = [None, yq, y2, y3]
                terms = []
                for n in range(4):
                    for m in range(4):
                        if xp[m] is None and yp[n] is None:
                            terms.append(None)
                        elif xp[m] is None:
                            terms.append(yp[n])
                        elif yp[n] is None:
                            terms.append(xp[m])
                        else:
                            terms.append(xp[m] * yp[n])
                rowv = iota16 + (s * 128 + l * 16)
                for p in range(NPROPS):
                    acc = None
                    for k in range(16):
                        colv = jnp.full((16,), p * 16 + k, jnp.int32)
                        cf = plsc.load_gather(rows, [rowv, colv])
                        if terms[k] is None:
                            acc = cf if acc is None else acc + cf
                        else:
                            acc = acc + cf * terms[k]
                    outs[p][s, pl.ds(l * 16, 16)] = acc
                return c
            lax.fori_loop(0, 8, qgrp, carry)

        for p in range(NPROPS):
            pltpu.sync_copy(outs[p],
                            out_hbm.at[pl.ds(p * PROWS + rowb, NSUB)])
        return carry

    lax.fori_loop(0, NCH, chunk, 0)


def _lookup_poly(idx2d, xf, yf, table):
    mesh = plsc.VectorSubcoreMesh(core_axis_name="c", subcore_axis_name="s")

    def body(idx_hbm, x_hbm, y_hbm, table_hbm, out_hbm, *scratch):
        xv, yv, idxv, rows = scratch[0], scratch[1], scratch[2], scratch[3]
        outs = list(scratch[4:4 + NPROPS])
        sem = scratch[-1]
        _sc_body(idx_hbm, x_hbm, y_hbm, table_hbm, out_hbm,
                 xv, yv, idxv, rows, outs, sem)

    scratch_types = (
        [pltpu.VMEM((NSUB, 128), jnp.float32),
         pltpu.VMEM((NSUB, 128), jnp.float32),
         pltpu.VMEM((NSUB, 128), jnp.int32),
         pltpu.VMEM((B, 128), jnp.float32)]
        + [pltpu.VMEM((NSUB, 128), jnp.float32) for _ in range(NPROPS)]
        + [pltpu.SemaphoreType.DMA]
    )
    fn = pl.kernel(
        body,
        out_type=jax.ShapeDtypeStruct((NPROPS * PROWS, 128), jnp.float32),
        mesh=mesh,
        compiler_params=pltpu.CompilerParams(needs_layout_passes=False,
                                             use_tc_tiling_on_sc=False),
        scratch_types=scratch_types,
    )
    return fn(idx2d, xf, yf, table)


def kernel(h, P, coeffs):
    h2 = h.reshape(PROWS, 128)
    p2 = P.reshape(PROWS, 128)
    table = _repack(coeffs)
    idx2, x2, y2 = _binning(h2, p2)
    out2d = _lookup_poly(idx2, x2, y2, table)
    return out2d.reshape(NPROPS, PROWS, 128).reshape(NPROPS, NPTS)


# speedup vs baseline: 2.8697x; 1.1237x over previous
import functools

import jax
import jax.numpy as jnp
from jax import lax
from jax.experimental import pallas as pl
from jax.experimental.pallas import tpu as pltpu
from jax.experimental.pallas import tpu_sc as plsc
import numpy as np

NPTS = 1048576
NGRID = 512
NPROPS = 5
HMIN, HMAX = 1.0e5, 4.0e6
PMIN, PMAX = 1.0e4, 1.0e7
LMIN, LMAX = float(np.log(PMIN)), float(np.log(PMAX))
NI = NGRID - 1
JSTRIDE = 512
TROWS = NI * JSTRIDE

NC, NS = 2, 16
NW = NC * NS
PW = NPTS // NW
PROWS = NPTS // 128
STG = 4096
NSTG = PW // STG
TSUB = STG // 128
RING = 4


def _repack_body(c_ref, t_ref):
    a = c_ref[...]
    parts = [a[p, 0] for p in range(NPROPS)]
    row = jnp.concatenate(parts, axis=-1)
    row = jnp.concatenate(
        [row, jnp.zeros((NI, 128 - 16 * NPROPS), jnp.float32)], axis=-1)
    row = jnp.concatenate([row, jnp.zeros((1, 128), jnp.float32)], axis=0)
    t_ref[...] = row


def _repack(coeffs):
    return pl.pallas_call(
        _repack_body,
        grid=(NI,),
        in_specs=[pl.BlockSpec((NPROPS, 1, NI, 16), lambda i: (0, i, 0, 0))],
        out_specs=pl.BlockSpec((JSTRIDE, 128), lambda i: (i, 0)),
        out_shape=jax.ShapeDtypeStruct((TROWS, 128), jnp.float32),
    )(coeffs)


_CH = float(np.float32(np.float32(1.0 / (HMAX - HMIN)) * np.float32(NGRID - 1)))
_CL = float(np.float32(np.float32(1.0 / (LMAX - LMIN)) * np.float32(NGRID - 1)))


def _bin_body(h_ref, p_ref, idx_ref, x_ref, y_ref):
    h = h_ref[...]
    ii = (h - HMIN) * _CH
    i = jnp.clip(ii.astype(jnp.int32), 0, NGRID - 2)
    x = ii - i.astype(ii.dtype)
    L = jnp.log(p_ref[...])
    jj = (L - LMIN) * _CL
    j = jnp.clip(jj.astype(jnp.int32), 0, NGRID - 2)
    y = jj - j.astype(jj.dtype)
    idx_ref[...] = i * JSTRIDE + j
    x_ref[...] = x
    y_ref[...] = y


def _binning(h2, p2):
    blk = pl.BlockSpec((1024, 128), lambda g: (g, 0))
    return pl.pallas_call(
        _bin_body,
        grid=(8,),
        in_specs=[blk, blk],
        out_specs=[blk, blk, blk],
        out_shape=[
            jax.ShapeDtypeStruct((PROWS, 128), jnp.int32),
            jax.ShapeDtypeStruct((PROWS, 128), jnp.float32),
            jax.ShapeDtypeStruct((PROWS, 128), jnp.float32),
        ],
    )(h2, p2)


def _sc_compute(rowsb, xv, yv, outs, t, iota16):
    def qgrp(l, c):
        xq = xv[t, pl.ds(l * 16, 16)]
        yq = yv[t, pl.ds(l * 16, 16)]
        x2 = xq * xq
        x3 = x2 * xq
        y2 = yq * yq
        y3 = y2 * yq
        xp = [None, xq, x2, x3]
        yp = [None, yq, y2, y3]
        terms = []
        for n in range(4):
            for m in range(4):
                if xp[m] is None and yp[n] is None:
                    terms.append(None)
                elif xp[m] is None:
                    terms.append(yp[n])
                elif yp[n] is None:
                    terms.append(xp[m])
                else:
                    terms.append(xp[m] * yp[n])
        rowv = iota16 + l * 16
        for p in range(NPROPS):
            acc = None
            for k in range(16):
                colv = jnp.full((16,), p * 16 + k, jnp.int32)
                cf = plsc.load_gather(rowsb, [rowv, colv])
                if terms[k] is None:
                    acc = cf if acc is None else acc + cf
                else:
                    acc = acc + cf * terms[k]
            outs[p][t, pl.ds(l * 16, 16)] = acc
        return c
    lax.fori_loop(0, 8, qgrp, 0)


def _sc_body(idx_hbm, x_hbm, y_hbm, table_hbm, out_hbm,
             xv, yv, idxv, rows, outs, sems):
    wid = lax.axis_index("s") * NC + lax.axis_index("c")
    iota16 = lax.iota(jnp.int32, 16)

    def stage(s, carry):
        rbase = wid * (PW // 128) + s * TSUB
        pltpu.sync_copy(x_hbm.at[pl.ds(rbase, TSUB)], xv)
        pltpu.sync_copy(y_hbm.at[pl.ds(rbase, TSUB)], yv)
        pltpu.sync_copy(idx_hbm.at[pl.ds(rbase, TSUB)], idxv)

        for b in range(RING - 1):
            pltpu.async_copy(table_hbm.at[idxv.at[b]], rows[b], sems[b])

        def quad(tq, c):
            for b in range(RING):
                t = tq * RING + b
                tf = t + (RING - 1)
                bf = (b + RING - 1) % RING

                @pl.when(tf < TSUB)
                def _():
                    pltpu.async_copy(table_hbm.at[idxv.at[tf]],
                                     rows[bf], sems[bf])

                pltpu.make_async_copy(table_hbm.at[idxv.at[t]],
                                      rows[b], sems[b]).wait()
                _sc_compute(rows[b], xv, yv, outs, t, iota16)
            return c
        lax.fori_loop(0, TSUB // RING, quad, 0)

        for p in range(NPROPS):
            pltpu.sync_copy(outs[p],
                            out_hbm.at[pl.ds(p * PROWS + rbase, TSUB)])
        return carry

    lax.fori_loop(0, NSTG, stage, 0)


def _lookup_poly(idx2d, xf, yf, table):
    mesh = plsc.VectorSubcoreMesh(core_axis_name="c", subcore_axis_name="s")

    def body(idx_hbm, x_hbm, y_hbm, table_hbm, out_hbm, *scratch):
        xv, yv, idxv = scratch[0], scratch[1], scratch[2]
        rows = list(scratch[3:3 + RING])
        outs = list(scratch[3 + RING:3 + RING + NPROPS])
        sems = list(scratch[3 + RING + NPROPS:3 + RING + NPROPS + RING])
        _sc_body(idx_hbm, x_hbm, y_hbm, table_hbm, out_hbm,
                 xv, yv, idxv, rows, outs, sems)

    scratch_types = (
        [pltpu.VMEM((TSUB, 128), jnp.float32),
         pltpu.VMEM((TSUB, 128), jnp.float32),
         pltpu.VMEM((TSUB, 128), jnp.int32)]
        + [pltpu.VMEM((128, 128), jnp.float32) for _ in range(RING)]
        + [pltpu.VMEM((TSUB, 128), jnp.float32) for _ in range(NPROPS)]
        + [pltpu.SemaphoreType.DMA for _ in range(RING)]
    )
    fn = pl.kernel(
        body,
        out_type=jax.ShapeDtypeStruct((NPROPS * PROWS, 128), jnp.float32),
        mesh=mesh,
        compiler_params=pltpu.CompilerParams(needs_layout_passes=False,
                                             use_tc_tiling_on_sc=False),
        scratch_types=scratch_types,
    )
    return fn(idx2d, xf, yf, table)


def kernel(h, P, coeffs):
    h2 = h.reshape(PROWS, 128)
    p2 = P.reshape(PROWS, 128)
    table = _repack(coeffs)
    idx2, x2, y2 = _binning(h2, p2)
    out2d = _lookup_poly(idx2, x2, y2, table)
    return out2d.reshape(NPROPS, PROWS, 128).reshape(NPROPS, NPTS)


# speedup vs baseline: 3.5058x; 1.2217x over previous
import functools

import jax
import jax.numpy as jnp
from jax import lax
from jax.experimental import pallas as pl
from jax.experimental.pallas import tpu as pltpu
from jax.experimental.pallas import tpu_sc as plsc
import numpy as np

NPTS = 1048576
NGRID = 512
NPROPS = 5
HMIN, HMAX = 1.0e5, 4.0e6
PMIN, PMAX = 1.0e4, 1.0e7
LMIN, LMAX = float(np.log(PMIN)), float(np.log(PMAX))
NI = NGRID - 1
JSTRIDE = 512
TROWS = NI * JSTRIDE

NC, NS = 2, 16
NW = NC * NS
PW = NPTS // NW
PROWS = NPTS // 128
STG = 4096
NSTG = PW // STG
TSUB = STG // 128
RING = 4
NSTRM = 4


def _repack_body(c_ref, t_ref):
    a = c_ref[...]
    parts = [jnp.swapaxes(a[p, 0], 0, 1) for p in range(NPROPS)]
    row = jnp.concatenate(parts, axis=-1)
    row = jnp.concatenate(
        [row, jnp.zeros((NI, 128 - 16 * NPROPS), jnp.float32)], axis=-1)
    row = jnp.concatenate([row, jnp.zeros((1, 128), jnp.float32)], axis=0)
    t_ref[...] = row


def _repack(coeffs):
    ct = jnp.transpose(coeffs, (0, 1, 3, 2))
    return pl.pallas_call(
        _repack_body,
        grid=(NI,),
        in_specs=[pl.BlockSpec((NPROPS, 1, 16, NI), lambda i: (0, i, 0, 0))],
        out_specs=pl.BlockSpec((JSTRIDE, 128), lambda i: (i, 0)),
        out_shape=jax.ShapeDtypeStruct((TROWS, 128), jnp.float32),
    )(ct)


_CH = float(np.float32(np.float32(1.0 / (HMAX - HMIN)) * np.float32(NGRID - 1)))
_CL = float(np.float32(np.float32(1.0 / (LMAX - LMIN)) * np.float32(NGRID - 1)))


def _bin_body(h_ref, p_ref, idx_ref, x_ref, y_ref):
    h = h_ref[...]
    ii = (h - HMIN) * _CH
    i = jnp.clip(ii.astype(jnp.int32), 0, NGRID - 2)
    x = ii - i.astype(ii.dtype)
    L = jnp.log(p_ref[...])
    jj = (L - LMIN) * _CL
    j = jnp.clip(jj.astype(jnp.int32), 0, NGRID - 2)
    y = jj - j.astype(jj.dtype)
    idx_ref[...] = i * JSTRIDE + j
    x_ref[...] = x
    y_ref[...] = y


def _binning(h2, p2):
    blk = pl.BlockSpec((1024, 128), lambda g: (g, 0))
    return pl.pallas_call(
        _bin_body,
        grid=(8,),
        in_specs=[blk, blk],
        out_specs=[blk, blk, blk],
        out_shape=[
            jax.ShapeDtypeStruct((PROWS, 128), jnp.int32),
            jax.ShapeDtypeStruct((PROWS, 128), jnp.float32),
            jax.ShapeDtypeStruct((PROWS, 128), jnp.float32),
        ],
    )(h2, p2)


def _sc_compute(rowsb, xv, yv, outs, t, iota16):
    def qgrp(l, c):
        xq = xv[t, pl.ds(l * 16, 16)]
        yq = yv[t, pl.ds(l * 16, 16)]
        x2 = xq * xq
        x3 = x2 * xq
        y2 = yq * yq
        y3 = y2 * yq
        xp = [None, xq, x2, x3]
        yp = [None, yq, y2, y3]
        terms = []
        for n in range(4):
            for m in range(4):
                if xp[m] is None and yp[n] is None:
                    terms.append(None)
                elif xp[m] is None:
                    terms.append(yp[n])
                elif yp[n] is None:
                    terms.append(xp[m])
                else:
                    terms.append(xp[m] * yp[n])
        rowv = iota16 + l * 16
        for p in range(NPROPS):
            acc = None
            for k in range(16):
                colv = jnp.full((16,), p * 16 + k, jnp.int32)
                cf = plsc.load_gather(rowsb, [rowv, colv])
                if terms[k] is None:
                    acc = cf if acc is None else acc + cf
                else:
                    acc = acc + cf * terms[k]
            outs[p][t, pl.ds(l * 16, 16)] = acc
        return c
    lax.fori_loop(0, 8, qgrp, 0)


def _sc_body(idx_hbm, x_hbm, y_hbm, table_hbm, out_hbm,
             xv, yv, idxv, rows, outs, sems):
    wid = lax.axis_index("s") * NC + lax.axis_index("c")
    iota16 = lax.iota(jnp.int32, 16)

    def stage(s, carry):
        rbase = wid * (PW // 128) + s * TSUB
        pltpu.sync_copy(x_hbm.at[pl.ds(rbase, TSUB)], xv)
        pltpu.sync_copy(y_hbm.at[pl.ds(rbase, TSUB)], yv)
        pltpu.sync_copy(idx_hbm.at[pl.ds(rbase, TSUB)], idxv)

        def fire(t, b):
            for u in range(NSTRM):
                sub = 128 // NSTRM
                pltpu.async_copy(
                    table_hbm.at[idxv.at[t, pl.ds(u * sub, sub)]],
                    rows[b].at[pl.ds(u * sub, sub)],
                    sems[b])

        def drain(t, b):
            for u in range(NSTRM):
                sub = 128 // NSTRM
                pltpu.make_async_copy(
                    table_hbm.at[idxv.at[t, pl.ds(u * sub, sub)]],
                    rows[b].at[pl.ds(u * sub, sub)],
                    sems[b]).wait()

        for b in range(RING - 1):
            fire(b, b)

        def quad(tq, c):
            for b in range(RING):
                t = tq * RING + b
                tf = t + (RING - 1)
                bf = (b + RING - 1) % RING

                @pl.when(tf < TSUB)
                def _():
                    fire(tf, bf)

                drain(t, b)
                _sc_compute(rows[b], xv, yv, outs, t, iota16)
            return c
        lax.fori_loop(0, TSUB // RING, quad, 0)

        for p in range(NPROPS):
            pltpu.sync_copy(outs[p],
                            out_hbm.at[pl.ds(p * PROWS + rbase, TSUB)])
        return carry

    lax.fori_loop(0, NSTG, stage, 0)


def _lookup_poly(idx2d, xf, yf, table):
    mesh = plsc.VectorSubcoreMesh(core_axis_name="c", subcore_axis_name="s")

    def body(idx_hbm, x_hbm, y_hbm, table_hbm, out_hbm, *scratch):
        xv, yv, idxv = scratch[0], scratch[1], scratch[2]
        rows = list(scratch[3:3 + RING])
        outs = list(scratch[3 + RING:3 + RING + NPROPS])
        sems = list(scratch[3 + RING + NPROPS:3 + RING + NPROPS + RING])
        _sc_body(idx_hbm, x_hbm, y_hbm, table_hbm, out_hbm,
                 xv, yv, idxv, rows, outs, sems)

    scratch_types = (
        [pltpu.VMEM((TSUB, 128), jnp.float32),
         pltpu.VMEM((TSUB, 128), jnp.float32),
         pltpu.VMEM((TSUB, 128), jnp.int32)]
        + [pltpu.VMEM((128, 128), jnp.float32) for _ in range(RING)]
        + [pltpu.VMEM((TSUB, 128), jnp.float32) for _ in range(NPROPS)]
        + [pltpu.SemaphoreType.DMA for _ in range(RING)]
    )
    fn = pl.kernel(
        body,
        out_type=jax.ShapeDtypeStruct((NPROPS * PROWS, 128), jnp.float32),
        mesh=mesh,
        compiler_params=pltpu.CompilerParams(needs_layout_passes=False,
                                             use_tc_tiling_on_sc=False),
        scratch_types=scratch_types,
    )
    return fn(idx2d, xf, yf, table)


def kernel(h, P, coeffs):
    h2 = h.reshape(PROWS, 128)
    p2 = P.reshape(PROWS, 128)
    table = _repack(coeffs)
    idx2, x2, y2 = _binning(h2, p2)
    out2d = _lookup_poly(idx2, x2, y2, table)
    return out2d.reshape(NPROPS, PROWS, 128).reshape(NPROPS, NPTS)


# speedup vs baseline: 5.1049x; 1.4561x over previous
import functools

import jax
import jax.numpy as jnp
from jax import lax
from jax.experimental import pallas as pl
from jax.experimental.pallas import tpu as pltpu
from jax.experimental.pallas import tpu_sc as plsc
import numpy as np

NPTS = 1048576
NGRID = 512
NPROPS = 5
HMIN, HMAX = 1.0e5, 4.0e6
PMIN, PMAX = 1.0e4, 1.0e7
LMIN, LMAX = float(np.log(PMIN)), float(np.log(PMAX))
NI = NGRID - 1
JSTRIDE = 512
TROWS = NI * JSTRIDE

NC, NS = 2, 16
NW = NC * NS
PW = NPTS // NW
PROWS = NPTS // 128
STG = 4096
NSTG = PW // STG
TSUB = STG // 128
RING = 4
NSTRM = 4


def _repack_body(c_ref, t_ref):
    a = c_ref[...]
    parts = [jnp.swapaxes(a[p, 0], 0, 1) for p in range(NPROPS)]
    row = jnp.concatenate(parts, axis=-1)
    row = jnp.concatenate(
        [row, jnp.zeros((NI, 128 - 16 * NPROPS), jnp.float32)], axis=-1)
    row = jnp.concatenate([row, jnp.zeros((1, 128), jnp.float32)], axis=0)
    t_ref[...] = row


def _repack(coeffs):
    ct = jnp.transpose(coeffs, (0, 1, 3, 2))
    return pl.pallas_call(
        _repack_body,
        grid=(NI,),
        in_specs=[pl.BlockSpec((NPROPS, 1, 16, NI), lambda i: (0, i, 0, 0))],
        out_specs=pl.BlockSpec((JSTRIDE, 128), lambda i: (i, 0)),
        out_shape=jax.ShapeDtypeStruct((TROWS, 128), jnp.float32),
    )(ct)


_CH = float(np.float32(np.float32(1.0 / (HMAX - HMIN)) * np.float32(NGRID - 1)))
_CL = float(np.float32(np.float32(1.0 / (LMAX - LMIN)) * np.float32(NGRID - 1)))


CROWS = TROWS // NW
CCH = 16
CB = CROWS // CCH


def _compress_body(t128_hbm, t80_hbm, buf0, buf1, semi0, semi1, semo0, semo1):
    wid = lax.axis_index("s") * NC + lax.axis_index("c")
    base = wid * CROWS
    bufs = (buf0, buf1)
    semi = (semi0, semi1)
    semo = (semo0, semo1)

    pltpu.async_copy(t128_hbm.at[pl.ds(base, CB)], bufs[0], semi[0])

    def pair(gg, c):
        for b in range(2):
            g = gg * 2 + b

            @pl.when(g >= 1)
            def _():
                pltpu.make_async_copy(
                    bufs[1 - b].at[:, pl.ds(0, 80)],
                    t80_hbm.at[pl.ds(base + (g - 1) * CB, CB)],
                    semo[1 - b]).wait()

            @pl.when(g + 1 < CCH)
            def _():
                pltpu.async_copy(t128_hbm.at[pl.ds(base + (g + 1) * CB, CB)],
                                 bufs[1 - b], semi[1 - b])

            pltpu.make_async_copy(t128_hbm.at[pl.ds(base + g * CB, CB)],
                                  bufs[b], semi[b]).wait()

            pltpu.async_copy(bufs[b].at[:, pl.ds(0, 80)],
                             t80_hbm.at[pl.ds(base + g * CB, CB)],
                             semo[b])
        return c
    lax.fori_loop(0, CCH // 2, pair, 0)

    pltpu.make_async_copy(bufs[1].at[:, pl.ds(0, 80)],
                          t80_hbm.at[pl.ds(base + (CCH - 1) * CB, CB)],
                          semo[1]).wait()


def _compress(table128):
    mesh = plsc.VectorSubcoreMesh(core_axis_name="c", subcore_axis_name="s")
    fn = pl.kernel(
        _compress_body,
        out_type=jax.ShapeDtypeStruct((TROWS, 80), jnp.float32),
        mesh=mesh,
        compiler_params=pltpu.CompilerParams(needs_layout_passes=False,
                                             use_tc_tiling_on_sc=False),
        scratch_types=[pltpu.VMEM((CB, 128), jnp.float32),
                       pltpu.VMEM((CB, 128), jnp.float32),
                       pltpu.SemaphoreType.DMA, pltpu.SemaphoreType.DMA,
                       pltpu.SemaphoreType.DMA, pltpu.SemaphoreType.DMA])
    return fn(table128)


def _bin_body(h_ref, p_ref, idx_ref, x_ref, y_ref):
    h = h_ref[...]
    ii = (h - HMIN) * _CH
    i = jnp.clip(ii.astype(jnp.int32), 0, NGRID - 2)
    x = ii - i.astype(ii.dtype)
    L = jnp.log(p_ref[...])
    jj = (L - LMIN) * _CL
    j = jnp.clip(jj.astype(jnp.int32), 0, NGRID - 2)
    y = jj - j.astype(jj.dtype)
    idx_ref[...] = i * JSTRIDE + j
    x_ref[...] = x
    y_ref[...] = y


def _binning(h2, p2):
    blk = pl.BlockSpec((1024, 128), lambda g: (g, 0))
    return pl.pallas_call(
        _bin_body,
        grid=(8,),
        in_specs=[blk, blk],
        out_specs=[blk, blk, blk],
        out_shape=[
            jax.ShapeDtypeStruct((PROWS, 128), jnp.int32),
            jax.ShapeDtypeStruct((PROWS, 128), jnp.float32),
            jax.ShapeDtypeStruct((PROWS, 128), jnp.float32),
        ],
    )(h2, p2)


def _sc_compute(rowsb, xv, yv, outs, t, iota16):
    def qgrp(l, c):
        xq = xv[t, pl.ds(l * 16, 16)]
        yq = yv[t, pl.ds(l * 16, 16)]
        x2 = xq * xq
        x3 = x2 * xq
        y2 = yq * yq
        y3 = y2 * yq
        xp = [None, xq, x2, x3]
        yp = [None, yq, y2, y3]
        terms = []
        for n in range(4):
            for m in range(4):
                if xp[m] is None and yp[n] is None:
                    terms.append(None)
                elif xp[m] is None:
                    terms.append(yp[n])
                elif yp[n] is None:
                    terms.append(xp[m])
                else:
                    terms.append(xp[m] * yp[n])
        rowv = iota16 + l * 16
        for p in range(NPROPS):
            acc = None
            for k in range(16):
                colv = jnp.full((16,), p * 16 + k, jnp.int32)
                cf = plsc.load_gather(rowsb, [rowv, colv])
                if terms[k] is None:
                    acc = cf if acc is None else acc + cf
                else:
                    acc = acc + cf * terms[k]
            outs[p][t, pl.ds(l * 16, 16)] = acc
        return c
    lax.fori_loop(0, 8, qgrp, 0)


def _sc_body(idx_hbm, x_hbm, y_hbm, table_hbm, out_hbm,
             xv, yv, idxv, rows, outs, sems):
    wid = lax.axis_index("s") * NC + lax.axis_index("c")
    iota16 = lax.iota(jnp.int32, 16)

    def stage(s, carry):
        rbase = wid * (PW // 128) + s * TSUB
        pltpu.sync_copy(x_hbm.at[pl.ds(rbase, TSUB)], xv)
        pltpu.sync_copy(y_hbm.at[pl.ds(rbase, TSUB)], yv)
        pltpu.sync_copy(idx_hbm.at[pl.ds(rbase, TSUB)], idxv)

        def fire(t, b):
            for u in range(NSTRM):
                sub = 128 // NSTRM
                pltpu.async_copy(
                    table_hbm.at[idxv.at[t, pl.ds(u * sub, sub)]],
                    rows[b].at[pl.ds(u * sub, sub)],
                    sems[b])

        def drain(t, b):
            for u in range(NSTRM):
                sub = 128 // NSTRM
                pltpu.make_async_copy(
                    table_hbm.at[idxv.at[t, pl.ds(u * sub, sub)]],
                    rows[b].at[pl.ds(u * sub, sub)],
                    sems[b]).wait()

        for b in range(RING - 1):
            fire(b, b)

        def quad(tq, c):
            for b in range(RING):
                t = tq * RING + b
                tf = t + (RING - 1)
                bf = (b + RING - 1) % RING

                @pl.when(tf < TSUB)
                def _():
                    fire(tf, bf)

                drain(t, b)
                _sc_compute(rows[b], xv, yv, outs, t, iota16)
            return c
        lax.fori_loop(0, TSUB // RING, quad, 0)

        for p in range(NPROPS):
            pltpu.sync_copy(outs[p],
                            out_hbm.at[pl.ds(p * PROWS + rbase, TSUB)])
        return carry

    lax.fori_loop(0, NSTG, stage, 0)


def _lookup_poly(idx2d, xf, yf, table):
    mesh = plsc.VectorSubcoreMesh(core_axis_name="c", subcore_axis_name="s")

    def body(idx_hbm, x_hbm, y_hbm, table_hbm, out_hbm, *scratch):
        xv, yv, idxv = scratch[0], scratch[1], scratch[2]
        rows = list(scratch[3:3 + RING])
        outs = list(scratch[3 + RING:3 + RING + NPROPS])
        sems = list(scratch[3 + RING + NPROPS:3 + RING + NPROPS + RING])
        _sc_body(idx_hbm, x_hbm, y_hbm, table_hbm, out_hbm,
                 xv, yv, idxv, rows, outs, sems)

    scratch_types = (
        [pltpu.VMEM((TSUB, 128), jnp.float32),
         pltpu.VMEM((TSUB, 128), jnp.float32),
         pltpu.VMEM((TSUB, 128), jnp.int32)]
        + [pltpu.VMEM((128, 80), jnp.float32) for _ in range(RING)]
        + [pltpu.VMEM((TSUB, 128), jnp.float32) for _ in range(NPROPS)]
        + [pltpu.SemaphoreType.DMA for _ in range(RING)]
    )
    fn = pl.kernel(
        body,
        out_type=jax.ShapeDtypeStruct((NPROPS * PROWS, 128), jnp.float32),
        mesh=mesh,
        compiler_params=pltpu.CompilerParams(needs_layout_passes=False,
                                             use_tc_tiling_on_sc=False),
        scratch_types=scratch_types,
    )
    return fn(idx2d, xf, yf, table)


def kernel(h, P, coeffs):
    h2 = h.reshape(PROWS, 128)
    p2 = P.reshape(PROWS, 128)
    table = _compress(_repack(coeffs))
    idx2, x2, y2 = _binning(h2, p2)
    out2d = _lookup_poly(idx2, x2, y2, table)
    return out2d.reshape(NPROPS, PROWS, 128).reshape(NPROPS, NPTS)


# speedup vs baseline: 6.1820x; 1.2110x over previous
import functools

import jax
import jax.numpy as jnp
from jax import lax
from jax.experimental import pallas as pl
from jax.experimental.pallas import tpu as pltpu
from jax.experimental.pallas import tpu_sc as plsc
import numpy as np

NPTS = 1048576
NGRID = 512
NPROPS = 5
HMIN, HMAX = 1.0e5, 4.0e6
PMIN, PMAX = 1.0e4, 1.0e7
LMIN, LMAX = float(np.log(PMIN)), float(np.log(PMAX))
NI = NGRID - 1
JSTRIDE = 512
TROWS = NI * JSTRIDE

NC, NS = 2, 16
NW = NC * NS
PW = NPTS // NW
PROWS = NPTS // 128
STG = 4096
NSTG = PW // STG
TSUB = STG // 128
RING = 8
NSTRM = 4


RSLAB = 7


def _repack_body(c_ref, t_ref):
    a = c_ref[...]
    parts = [jnp.swapaxes(a[p].reshape(RSLAB * 16, NI), 0, 1)
             for p in range(NPROPS)]
    zc = jnp.zeros((NI, 128 - 16 * NPROPS), jnp.float32)
    zr = jnp.zeros((1, 128), jnp.float32)
    slabs = []
    for ii in range(RSLAB):
        row = jnp.concatenate(
            [parts[p][:, ii * 16:(ii + 1) * 16] for p in range(NPROPS)]
            + [zc], axis=-1)
        slabs.append(jnp.concatenate([row, zr], axis=0))
    t_ref[...] = jnp.concatenate(slabs, axis=0)


def _repack(coeffs):
    ct = jnp.transpose(coeffs, (0, 1, 3, 2))
    return pl.pallas_call(
        _repack_body,
        grid=(NI // RSLAB,),
        in_specs=[pl.BlockSpec((NPROPS, RSLAB, 16, NI),
                               lambda i: (0, i, 0, 0))],
        out_specs=pl.BlockSpec((RSLAB * JSTRIDE, 128), lambda i: (i, 0)),
        out_shape=jax.ShapeDtypeStruct((TROWS, 128), jnp.float32),
    )(ct)


_CH = float(np.float32(np.float32(1.0 / (HMAX - HMIN)) * np.float32(NGRID - 1)))
_CL = float(np.float32(np.float32(1.0 / (LMAX - LMIN)) * np.float32(NGRID - 1)))


CROWS = TROWS // NW
CCH = 16
CB = CROWS // CCH


def _compress_body(t128_hbm, t80_hbm, buf0, buf1, semi0, semi1, semo0, semo1):
    wid = lax.axis_index("s") * NC + lax.axis_index("c")
    base = wid * CROWS
    bufs = (buf0, buf1)
    semi = (semi0, semi1)
    semo = (semo0, semo1)

    pltpu.async_copy(t128_hbm.at[pl.ds(base, CB)], bufs[0], semi[0])

    def pair(gg, c):
        for b in range(2):
            g = gg * 2 + b

            @pl.when(g >= 1)
            def _():
                pltpu.make_async_copy(
                    bufs[1 - b].at[:, pl.ds(0, 80)],
                    t80_hbm.at[pl.ds(base + (g - 1) * CB, CB)],
                    semo[1 - b]).wait()

            @pl.when(g + 1 < CCH)
            def _():
                pltpu.async_copy(t128_hbm.at[pl.ds(base + (g + 1) * CB, CB)],
                                 bufs[1 - b], semi[1 - b])

            pltpu.make_async_copy(t128_hbm.at[pl.ds(base + g * CB, CB)],
                                  bufs[b], semi[b]).wait()

            pltpu.async_copy(bufs[b].at[:, pl.ds(0, 80)],
                             t80_hbm.at[pl.ds(base + g * CB, CB)],
                             semo[b])
        return c
    lax.fori_loop(0, CCH // 2, pair, 0)

    pltpu.make_async_copy(bufs[1].at[:, pl.ds(0, 80)],
                          t80_hbm.at[pl.ds(base + (CCH - 1) * CB, CB)],
                          semo[1]).wait()


def _compress(table128):
    mesh = plsc.VectorSubcoreMesh(core_axis_name="c", subcore_axis_name="s")
    fn = pl.kernel(
        _compress_body,
        out_type=jax.ShapeDtypeStruct((TROWS, 80), jnp.float32),
        mesh=mesh,
        compiler_params=pltpu.CompilerParams(needs_layout_passes=False,
                                             use_tc_tiling_on_sc=False),
        scratch_types=[pltpu.VMEM((CB, 128), jnp.float32),
                       pltpu.VMEM((CB, 128), jnp.float32),
                       pltpu.SemaphoreType.DMA, pltpu.SemaphoreType.DMA,
                       pltpu.SemaphoreType.DMA, pltpu.SemaphoreType.DMA])
    return fn(table128)


def _bin_body(h_ref, p_ref, idx_ref, x_ref, y_ref):
    h = h_ref[...]
    ii = (h - HMIN) * _CH
    i = jnp.clip(ii.astype(jnp.int32), 0, NGRID - 2)
    x = ii - i.astype(ii.dtype)
    L = jnp.log(p_ref[...])
    jj = (L - LMIN) * _CL
    j = jnp.clip(jj.astype(jnp.int32), 0, NGRID - 2)
    y = jj - j.astype(jj.dtype)
    idx_ref[...] = i * JSTRIDE + j
    x_ref[...] = x
    y_ref[...] = y


def _binning(h2, p2):
    blk = pl.BlockSpec((1024, 128), lambda g: (g, 0))
    return pl.pallas_call(
        _bin_body,
        grid=(8,),
        in_specs=[blk, blk],
        out_specs=[blk, blk, blk],
        out_shape=[
            jax.ShapeDtypeStruct((PROWS, 128), jnp.int32),
            jax.ShapeDtypeStruct((PROWS, 128), jnp.float32),
            jax.ShapeDtypeStruct((PROWS, 128), jnp.float32),
        ],
    )(h2, p2)


def _sc_compute(rowsb, xv, yv, outs, t, iota16):
    def qgrp(l, c):
        xq = xv[t, pl.ds(l * 16, 16)]
        yq = yv[t, pl.ds(l * 16, 16)]
        x2 = xq * xq
        x3 = x2 * xq
        y2 = yq * yq
        y3 = y2 * yq
        xp = [None, xq, x2, x3]
        yp = [None, yq, y2, y3]
        terms = []
        for n in range(4):
            for m in range(4):
                if xp[m] is None and yp[n] is None:
                    terms.append(None)
                elif xp[m] is None:
                    terms.append(yp[n])
                elif yp[n] is None:
                    terms.append(xp[m])
                else:
                    terms.append(xp[m] * yp[n])
        rowv = iota16 + l * 16
        for p in range(NPROPS):
            acc = None
            for k in range(16):
                colv = jnp.full((16,), p * 16 + k, jnp.int32)
                cf = plsc.load_gather(rowsb, [rowv, colv])
                if terms[k] is None:
                    acc = cf if acc is None else acc + cf
                else:
                    acc = acc + cf * terms[k]
            outs[p][t, pl.ds(l * 16, 16)] = acc
        return c
    lax.fori_loop(0, 8, qgrp, 0)


def _sc_body(idx_hbm, x_hbm, y_hbm, table_hbm, out_hbm,
             xv, yv, idxv, rows, outs, sems):
    wid = lax.axis_index("s") * NC + lax.axis_index("c")
    iota16 = lax.iota(jnp.int32, 16)

    def stage(s, carry):
        rbase = wid * (PW // 128) + s * TSUB
        pltpu.sync_copy(x_hbm.at[pl.ds(rbase, TSUB)], xv)
        pltpu.sync_copy(y_hbm.at[pl.ds(rbase, TSUB)], yv)
        pltpu.sync_copy(idx_hbm.at[pl.ds(rbase, TSUB)], idxv)

        def fire(t, b):
            for u in range(NSTRM):
                sub = 128 // NSTRM
                pltpu.async_copy(
                    table_hbm.at[idxv.at[t, pl.ds(u * sub, sub)]],
                    rows[b].at[pl.ds(u * sub, sub)],
                    sems[b])

        def drain(t, b):
            for u in range(NSTRM):
                sub = 128 // NSTRM
                pltpu.make_async_copy(
                    table_hbm.at[idxv.at[t, pl.ds(u * sub, sub)]],
                    rows[b].at[pl.ds(u * sub, sub)],
                    sems[b]).wait()

        for b in range(RING - 1):
            fire(b, b)

        def quad(tq, c):
            for b in range(RING):
                t = tq * RING + b
                tf = t + (RING - 1)
                bf = (b + RING - 1) % RING

                @pl.when(tf < TSUB)
                def _():
                    fire(tf, bf)

                drain(t, b)
                _sc_compute(rows[b], xv, yv, outs, t, iota16)
            return c
        lax.fori_loop(0, TSUB // RING, quad, 0)

        for p in range(NPROPS):
            pltpu.sync_copy(outs[p],
                            out_hbm.at[pl.ds(p * PROWS + rbase, TSUB)])
        return carry

    lax.fori_loop(0, NSTG, stage, 0)


def _lookup_poly(idx2d, xf, yf, table):
    mesh = plsc.VectorSubcoreMesh(core_axis_name="c", subcore_axis_name="s")

    def body(idx_hbm, x_hbm, y_hbm, table_hbm, out_hbm, *scratch):
        xv, yv, idxv = scratch[0], scratch[1], scratch[2]
        rows = list(scratch[3:3 + RING])
        outs = list(scratch[3 + RING:3 + RING + NPROPS])
        sems = list(scratch[3 + RING + NPROPS:3 + RING + NPROPS + RING])
        _sc_body(idx_hbm, x_hbm, y_hbm, table_hbm, out_hbm,
                 xv, yv, idxv, rows, outs, sems)

    scratch_types = (
        [pltpu.VMEM((TSUB, 128), jnp.float32),
         pltpu.VMEM((TSUB, 128), jnp.float32),
         pltpu.VMEM((TSUB, 128), jnp.int32)]
        + [pltpu.VMEM((128, 80), jnp.float32) for _ in range(RING)]
        + [pltpu.VMEM((TSUB, 128), jnp.float32) for _ in range(NPROPS)]
        + [pltpu.SemaphoreType.DMA for _ in range(RING)]
    )
    fn = pl.kernel(
        body,
        out_type=jax.ShapeDtypeStruct((NPROPS * PROWS, 128), jnp.float32),
        mesh=mesh,
        compiler_params=pltpu.CompilerParams(needs_layout_passes=False,
                                             use_tc_tiling_on_sc=False),
        scratch_types=scratch_types,
    )
    return fn(idx2d, xf, yf, table)


def kernel(h, P, coeffs):
    h2 = h.reshape(PROWS, 128)
    p2 = P.reshape(PROWS, 128)
    table = _compress(_repack(coeffs))
    idx2, x2, y2 = _binning(h2, p2)
    out2d = _lookup_poly(idx2, x2, y2, table)
    return out2d.reshape(NPROPS, PROWS, 128).reshape(NPROPS, NPTS)
